# baseline (speedup 1.0000x reference)
import jax
import jax.numpy as jnp
from jax.experimental import pallas as pl
from jax.experimental.pallas import tpu as pltpu

K = 100000
DIM = 128
B = 16384

_CHUNKS = ((0, B), (B, 20904), (B + 20904, 20904), (B + 41808, 20904),
           (B + 62712, K - B - 62712))


def _push_body(bank_ref, value_ref, out_ref, *scratch):
    n = len(_CHUNKS)
    bufs, sin, sout = scratch[:n], scratch[n:2 * n], scratch[2 * n:]
    ins, outs = [], []
    for i, (r0, nr) in enumerate(_CHUNKS):
        src = value_ref.at[pl.ds(0, nr)] if i == 0 else bank_ref.at[pl.ds(r0, nr)]
        ins.append(pltpu.make_async_copy(src, bufs[i], sin[i]))
        outs.append(pltpu.make_async_copy(
            bufs[i], out_ref.at[pl.ds(r0, nr)], sout[i]))
    for c in ins:
        c.start()
    for i in range(n):
        ins[i].wait()
        outs[i].start()
    for c in outs:
        c.wait()


@jax.jit
def kernel(bank, value):
    return pl.pallas_call(
        _push_body,
        out_shape=jax.ShapeDtypeStruct((K, DIM), jnp.float32),
        in_specs=[
            pl.BlockSpec(memory_space=pl.ANY),
            pl.BlockSpec(memory_space=pl.ANY),
        ],
        out_specs=pl.BlockSpec(memory_space=pl.ANY),
        scratch_shapes=(
            [pltpu.VMEM((nr, DIM), jnp.float32) for _, nr in _CHUNKS]
            + [pltpu.SemaphoreType.DMA] * (2 * len(_CHUNKS))
        ),
    )(bank, value)


# speedup vs baseline: 1.0348x; 1.0348x over previous
import jax
import jax.numpy as jnp
from jax.experimental import pallas as pl
from jax.experimental.pallas import tpu as pltpu

K = 100000
DIM = 128
B = 16384

_CH = 32768
_NCH = (K + _CH - 1) // _CH
_NBUF = 3


def _rows(i):
    return min(_CH, K - i * _CH)


def _in_copies(i, bank_ref, value_ref, buf, sem):
    r0, r1 = i * _CH, i * _CH + _rows(i)
    copies = []
    if r0 < B:
        n = min(r1, B) - r0
        copies.append(pltpu.make_async_copy(
            value_ref.at[pl.ds(r0, n)], buf.at[pl.ds(0, n)], sem))
    if r1 > B:
        s = max(r0, B)
        n = r1 - s
        copies.append(pltpu.make_async_copy(
            bank_ref.at[pl.ds(s, n)], buf.at[pl.ds(s - r0, n)], sem))
    return copies


def _push_body(bank_ref, value_ref, out_ref, *scratch):
    bufs, sin, sout = scratch[:_NBUF], scratch[_NBUF:2 * _NBUF], scratch[2 * _NBUF:]
    ins = [_in_copies(i, bank_ref, value_ref, bufs[i % _NBUF], sin[i % _NBUF])
           for i in range(_NCH)]
    outs = [pltpu.make_async_copy(
        bufs[i % _NBUF].at[pl.ds(0, _rows(i))],
        out_ref.at[pl.ds(i * _CH, _rows(i))], sout[i % _NBUF])
        for i in range(_NCH)]
    for i in range(_NBUF):
        if i < _NCH:
            for c in ins[i]:
                c.start()
    for i in range(_NCH):
        for c in ins[i]:
            c.wait()
        outs[i].start()
        if i + _NBUF < _NCH:
            outs[i].wait()
            for c in ins[i + _NBUF]:
                c.start()
    for i in range(max(0, _NCH - _NBUF), _NCH):
        outs[i].wait()


@jax.jit
def kernel(bank, value):
    return pl.pallas_call(
        _push_body,
        out_shape=jax.ShapeDtypeStruct((K, DIM), jnp.float32),
        in_specs=[
            pl.BlockSpec(memory_space=pl.ANY),
            pl.BlockSpec(memory_space=pl.ANY),
        ],
        out_specs=pl.BlockSpec(memory_space=pl.ANY),
        scratch_shapes=(
            [pltpu.VMEM((_CH, DIM), jnp.float32)] * _NBUF
            + [pltpu.SemaphoreType.DMA] * (2 * _NBUF)
        ),
    )(bank, value)


# speedup vs baseline: 1.0360x; 1.0012x over previous
import jax
import jax.numpy as jnp
from jax.experimental import pallas as pl
from jax.experimental.pallas import tpu as pltpu

K = 100000
DIM = 128
B = 16384

_CHUNKS = ((0, B), (B, 32768), (B + 32768, 32768), (B + 65536, K - B - 65536))


def _push_body(bank_ref, value_ref, out_ref, *scratch):
    n = len(_CHUNKS)
    bufs, sin, sout = scratch[:n], scratch[n:2 * n], scratch[2 * n:]
    ins, outs = [], []
    for i, (r0, nr) in enumerate(_CHUNKS):
        src = value_ref.at[pl.ds(0, nr)] if i == 0 else bank_ref.at[pl.ds(r0, nr)]
        ins.append(pltpu.make_async_copy(src, bufs[i], sin[i]))
        outs.append(pltpu.make_async_copy(
            bufs[i], out_ref.at[pl.ds(r0, nr)], sout[i]))
    for c in ins:
        c.start()
    for i in range(n):
        ins[i].wait()
        outs[i].start()
    for c in outs:
        c.wait()


@jax.jit
def kernel(bank, value):
    return pl.pallas_call(
        _push_body,
        out_shape=jax.ShapeDtypeStruct((K, DIM), jnp.float32),
        in_specs=[
            pl.BlockSpec(memory_space=pl.ANY),
            pl.BlockSpec(memory_space=pl.ANY),
        ],
        out_specs=pl.BlockSpec(memory_space=pl.ANY),
        scratch_shapes=(
            [pltpu.VMEM((nr, DIM), jnp.float32) for _, nr in _CHUNKS]
            + [pltpu.SemaphoreType.DMA] * (2 * len(_CHUNKS))
        ),
    )(bank, value)
